# Initial kernel scaffold; baseline (speedup 1.0000x reference)
import jax
import jax.numpy as jnp
from jax.experimental import pallas as pl


def kernel(embedded_state, batch_index, state_index, Wa, Wd):
    raise NotImplementedError("write your pallas kernel here")


# speedup vs baseline: 2.8565x; 2.8565x over previous
import functools

import jax
import jax.numpy as jnp
from jax import lax
from jax.experimental import pallas as pl
from jax.experimental.pallas import tpu as pltpu

DIM = 128
NACT = 8
NSEG = 16
TOTAL = 32768
BLK = 2048
NB = TOTAL // BLK

_NEG_INF = float("-inf")


def _proj_stats_body(x_ref, w_ref, seg_ref, la_ref, d_ref, c_ref, m_s, s_s):
    i = pl.program_id(0)

    @pl.when(i == 0)
    def _init():
        m_s[...] = jnp.full((NSEG, 1), _NEG_INF, jnp.float32)
        s_s[...] = jnp.zeros((NSEG, 1), jnp.float32)

    yt = lax.dot_general(
        w_ref[...], x_ref[...], (((1,), (1,)), ((), ())),
        preferred_element_type=jnp.float32,
    ) * (1.0 / DIM)

    d_row = yt[NACT:NACT + 1, :]
    d_ref[...] = d_row

    a = yt[:NACT, :]
    m8 = jnp.max(a, axis=0, keepdims=True)
    lse = jnp.log(jnp.sum(jnp.exp(a - m8), axis=0, keepdims=True))
    la_ref[...] = a - m8 - lse

    segr = seg_ref[0]
    onehot = lax.broadcasted_iota(jnp.int32, (NSEG, BLK), 0) == segr
    d_b = jnp.broadcast_to(d_row, (NSEG, BLK))
    dm = jnp.where(onehot, d_b, _NEG_INF)
    bmax = jnp.max(dm, axis=1, keepdims=True)

    m_old = m_s[...]
    s_old = s_s[...]
    m_new = jnp.maximum(m_old, bmax)
    scale = jnp.where(m_old == m_new, 1.0, jnp.exp(m_old - m_new))
    econ = jnp.where(onehot, jnp.exp(d_b - m_new), 0.0)
    s_new = s_old * scale + jnp.sum(econ, axis=1, keepdims=True)
    m_s[...] = m_new
    s_s[...] = s_new

    m_cl = jnp.where(jnp.isfinite(m_new), m_new, 0.0)
    c_ref[...] = m_cl + jnp.log(s_new + 1e-12)


def _apply_body(la_ref, d_ref, seg_ref, c_ref, out_ref):
    segr = seg_ref[0]
    onehot = lax.broadcasted_iota(jnp.int32, (NSEG, BLK), 0) == segr
    c_b = jnp.broadcast_to(c_ref[...], (NSEG, BLK))
    cg = jnp.sum(jnp.where(onehot, c_b, 0.0), axis=0, keepdims=True)
    ld = d_ref[...] - cg
    o_t = la_ref[...] + ld
    out_ref[...] = o_t.T


@jax.jit
def kernel(embedded_state, batch_index, state_index, Wa, Wd):
    del state_index
    x = embedded_state
    seg = batch_index.astype(jnp.int32)
    w = jnp.zeros((NSEG, DIM), jnp.float32)
    w = w.at[:NACT].set(Wa).at[NACT].set(Wd[0])
    seg3 = seg.reshape(NB, 1, BLK)

    la_t, d_t, c = pl.pallas_call(
        _proj_stats_body,
        grid=(NB,),
        in_specs=[
            pl.BlockSpec((BLK, DIM), lambda i: (i, 0)),
            pl.BlockSpec((NSEG, DIM), lambda i: (0, 0)),
            pl.BlockSpec((1, 1, BLK), lambda i: (i, 0, 0)),
        ],
        out_specs=[
            pl.BlockSpec((NACT, BLK), lambda i: (0, i)),
            pl.BlockSpec((1, BLK), lambda i: (0, i)),
            pl.BlockSpec((NSEG, 1), lambda i: (0, 0)),
        ],
        out_shape=[
            jax.ShapeDtypeStruct((NACT, TOTAL), jnp.float32),
            jax.ShapeDtypeStruct((1, TOTAL), jnp.float32),
            jax.ShapeDtypeStruct((NSEG, 1), jnp.float32),
        ],
        scratch_shapes=[
            pltpu.VMEM((NSEG, 1), jnp.float32),
            pltpu.VMEM((NSEG, 1), jnp.float32),
        ],
    )(x, w, seg3)

    out = pl.pallas_call(
        _apply_body,
        grid=(NB,),
        in_specs=[
            pl.BlockSpec((NACT, BLK), lambda i: (0, i)),
            pl.BlockSpec((1, BLK), lambda i: (0, i)),
            pl.BlockSpec((1, 1, BLK), lambda i: (i, 0, 0)),
            pl.BlockSpec((NSEG, 1), lambda i: (0, 0)),
        ],
        out_specs=pl.BlockSpec((BLK, NACT), lambda i: (i, 0)),
        out_shape=jax.ShapeDtypeStruct((TOTAL, NACT), jnp.float32),
    )(la_t, d_t, seg3, c)
    return out
